# Initial kernel scaffold; baseline (speedup 1.0000x reference)
import jax
import jax.numpy as jnp
from jax.experimental import pallas as pl


def kernel(x, embedding):
    raise NotImplementedError("write your pallas kernel here")


# speedup vs baseline: 1.2678x; 1.2678x over previous
import jax
import jax.numpy as jnp
from jax.experimental import pallas as pl
from jax.experimental.pallas import tpu as pltpu
from jax.experimental.pallas import tpu_sc as plsc

_NUM_CODES = 8192
_DIM = 256
_TOKENS = 8192
_TM = 256
_NTILES = _TOKENS // _TM
_GATHER_W = 128


def _dist_argmin_kernel(x_ref, embT_ref, idx_ref, loss_ref, esq_ref):
    i = pl.program_id(0)

    @pl.when(i == 0)
    def _init():
        e = embT_ref[...]
        esq_ref[...] = jnp.sum(e * e, axis=0, keepdims=True)
        loss_ref[...] = jnp.zeros_like(loss_ref)

    x = x_ref[...]
    xsq = jnp.sum(x * x, axis=1, keepdims=True)
    mm = jax.lax.dot_general(
        x, embT_ref[...],
        dimension_numbers=(((1,), (0,)), ((), ())),
        preferred_element_type=jnp.float32,
    )
    dist = (xsq - 2.0 * mm) + esq_ref[...]
    m = jnp.min(dist, axis=1, keepdims=True)
    ids = jnp.where(dist == m,
                    jax.lax.broadcasted_iota(jnp.int32, dist.shape, 1),
                    jnp.int32(_NUM_CODES))
    idx_ref[...] = jnp.min(ids, axis=1).reshape(1, 1, _TM)
    loss_ref[...] += jnp.sum(m, keepdims=True).reshape(1, 1)

    @pl.when(i == _NTILES - 1)
    def _finalize():
        loss_ref[...] = loss_ref[...] * (0.25 / float(_TOKENS * _DIM))


def _vq_tc(flat_x, embT):
    return pl.pallas_call(
        _dist_argmin_kernel,
        grid=(_NTILES,),
        in_specs=[
            pl.BlockSpec((_TM, _DIM), lambda i: (i, 0)),
            pl.BlockSpec((_DIM, _NUM_CODES), lambda i: (0, 0)),
        ],
        out_specs=[
            pl.BlockSpec((1, 1, _TM), lambda i: (i, 0, 0)),
            pl.BlockSpec((1, 1), lambda i: (0, 0)),
        ],
        out_shape=[
            jax.ShapeDtypeStruct((_NTILES, 1, _TM), jnp.int32),
            jax.ShapeDtypeStruct((1, 1), jnp.float32),
        ],
        scratch_shapes=[pltpu.VMEM((1, _NUM_CODES), jnp.float32)],
    )(flat_x, embT)


def _sc_gather(emb, idx2d):
    @pl.kernel(
        out_type=jax.ShapeDtypeStruct((_TOKENS, _DIM), emb.dtype),
        mesh=plsc.VectorSubcoreMesh(core_axis_name="core",
                                    subcore_axis_name="subcore"),
    )
    def _gather(x_hbm, i_hbm, o_hbm):
        def body(i_vmem, o_vmem):
            pltpu.sync_copy(x_hbm.at[i_vmem.at[0]], o_vmem)

        pltpu.emit_pipeline(
            body,
            grid=(_TOKENS // _GATHER_W,),
            in_specs=[pl.BlockSpec((1, _GATHER_W), index_map=lambda i: (0, i))],
            out_specs=[pl.BlockSpec((_GATHER_W, _DIM),
                                    index_map=lambda i: (i, 0))],
            core_axis_name=("core", "subcore"),
            dimension_semantics=(pltpu.PARALLEL,),
        )(i_hbm, o_hbm)

    return _gather(emb, idx2d)


def kernel(x, embedding):
    B, C, H, W = x.shape
    flat_x = jnp.transpose(x, (0, 2, 3, 1)).reshape(-1, C)
    embT = embedding.T
    idx3, loss11 = _vq_tc(flat_x, embT)
    idx = idx3.reshape(-1)
    q_flat = _sc_gather(embedding, idx.reshape(1, -1))
    quantized = jnp.transpose(q_flat.reshape(B, H, W, C), (0, 3, 1, 2))
    return (loss11[0, 0], quantized, idx)


# speedup vs baseline: 1.3422x; 1.0587x over previous
import jax
import jax.numpy as jnp
from jax.experimental import pallas as pl
from jax.experimental.pallas import tpu as pltpu
from jax.experimental.pallas import tpu_sc as plsc

_NUM_CODES = 8192
_DIM = 256
_TOKENS = 8192
_TM = 256
_NTILES = _TOKENS // _TM
_GATHER_W = 128


def _dist_argmin_kernel(x_ref, embT_ref, idx_ref, loss_ref, esq_ref, iota_ref):
    i = pl.program_id(0)

    @pl.when(i == 0)
    def _init():
        e = embT_ref[...]
        esq_ref[...] = jnp.sum(e * e, axis=0, keepdims=True)
        iota_ref[...] = jax.lax.broadcasted_iota(
            jnp.int32, (1, _NUM_CODES), 1).astype(jnp.float32)
        loss_ref[...] = jnp.zeros_like(loss_ref)

    x = x_ref[...]
    xsq = jnp.sum(x * x, axis=1, keepdims=True)
    mm = jax.lax.dot_general(
        x, embT_ref[...],
        dimension_numbers=(((1,), (0,)), ((), ())),
        preferred_element_type=jnp.float32,
    )
    dist = (xsq - 2.0 * mm) + esq_ref[...]
    m = jnp.min(dist, axis=1, keepdims=True)
    ids = jnp.where(dist == m, iota_ref[...], jnp.float32(_NUM_CODES))
    idx_ref[...] = jnp.min(ids, axis=1).astype(jnp.int32).reshape(1, 1, _TM)
    loss_ref[...] += jnp.sum(m, keepdims=True).reshape(1, 1)

    @pl.when(i == _NTILES - 1)
    def _finalize():
        loss_ref[...] = loss_ref[...] * (0.25 / float(_TOKENS * _DIM))


def _vq_tc(flat_x, embT):
    return pl.pallas_call(
        _dist_argmin_kernel,
        grid=(_NTILES,),
        in_specs=[
            pl.BlockSpec((_TM, _DIM), lambda i: (i, 0)),
            pl.BlockSpec((_DIM, _NUM_CODES), lambda i: (0, 0)),
        ],
        out_specs=[
            pl.BlockSpec((1, 1, _TM), lambda i: (i, 0, 0)),
            pl.BlockSpec((1, 1), lambda i: (0, 0)),
        ],
        out_shape=[
            jax.ShapeDtypeStruct((_NTILES, 1, _TM), jnp.int32),
            jax.ShapeDtypeStruct((1, 1), jnp.float32),
        ],
        scratch_shapes=[pltpu.VMEM((1, _NUM_CODES), jnp.float32),
                        pltpu.VMEM((1, _NUM_CODES), jnp.float32)],
    )(flat_x, embT)


def _sc_gather(emb, idx2d):
    @pl.kernel(
        out_type=jax.ShapeDtypeStruct((_TOKENS, _DIM), emb.dtype),
        mesh=plsc.VectorSubcoreMesh(core_axis_name="core",
                                    subcore_axis_name="subcore"),
    )
    def _gather(x_hbm, i_hbm, o_hbm):
        def body(i_vmem, o_vmem):
            pltpu.sync_copy(x_hbm.at[i_vmem.at[0]], o_vmem)

        pltpu.emit_pipeline(
            body,
            grid=(_TOKENS // _GATHER_W,),
            in_specs=[pl.BlockSpec((1, _GATHER_W), index_map=lambda i: (0, i))],
            out_specs=[pl.BlockSpec((_GATHER_W, _DIM),
                                    index_map=lambda i: (i, 0))],
            core_axis_name=("core", "subcore"),
            dimension_semantics=(pltpu.PARALLEL,),
        )(i_hbm, o_hbm)

    return _gather(emb, idx2d)


def kernel(x, embedding):
    B, C, H, W = x.shape
    flat_x = jnp.transpose(x, (0, 2, 3, 1)).reshape(-1, C)
    embT = embedding.T
    idx3, loss11 = _vq_tc(flat_x, embT)
    idx = idx3.reshape(-1)
    q_flat = _sc_gather(embedding, idx.reshape(1, -1))
    quantized = jnp.transpose(q_flat.reshape(B, H, W, C), (0, 3, 1, 2))
    return (loss11[0, 0], quantized, idx)
